# Initial kernel scaffold; baseline (speedup 1.0000x reference)
import functools

import jax
import jax.numpy as jnp
from jax import lax
from jax.experimental import pallas as pl
from jax.experimental.pallas import tpu as pltpu
from jax.experimental.pallas import tpu_sc as plsc

BATCH = 16384
NUM_FEATURES = 100
NUM_BUCKETS = 20
EMBED_DIM = 32
TOTAL = BATCH * NUM_FEATURES

NUM_WORKERS = 32
PER_W = TOTAL // NUM_WORKERS
CHUNK = 1024
NCHUNKS = PER_W // CHUNK


def _bucketize_body(x_ref, b_ref, out_ref):
    x = x_ref[...]
    x_min = jnp.min(x, axis=0, keepdims=True)
    x_max = jnp.max(x, axis=0, keepdims=True)
    xn = (x - x_min) / (x_max - x_min + 1e-08)
    bs = jax.nn.sigmoid(b_ref[...])
    cnt = (xn > 0.0).astype(jnp.int32)
    for k in range(NUM_BUCKETS - 1):
        cnt = cnt + (xn > bs[0:1, k : k + 1]).astype(jnp.int32)
    idx = jnp.clip(cnt - 1, 0, NUM_BUCKETS - 1)
    feat = lax.broadcasted_iota(jnp.int32, (BATCH, NUM_FEATURES), 1)
    out_ref[...] = idx + feat * NUM_BUCKETS


def _flat_indices(x, boundaries):
    return pl.pallas_call(
        _bucketize_body,
        out_shape=jax.ShapeDtypeStruct((BATCH, NUM_FEATURES), jnp.int32),
    )(x, boundaries.reshape(1, NUM_BUCKETS - 1))


def _sc_gather_body(table_hbm, idx_hbm, out_hbm, idx_v, rows_v, sem):
    wid = lax.axis_index("s") * 2 + lax.axis_index("c")
    base = wid * PER_W

    def body(c, carry):
        off = base + c * CHUNK
        pltpu.sync_copy(idx_hbm.at[pl.ds(off, CHUNK)], idx_v)
        pltpu.async_copy(table_hbm.at[idx_v], rows_v, sem).wait()
        pltpu.sync_copy(rows_v, out_hbm.at[pl.ds(off, CHUNK)])
        return carry

    lax.fori_loop(0, NCHUNKS, body, 0)


_sc_gather = functools.partial(
    pl.kernel,
    out_type=jax.ShapeDtypeStruct((TOTAL, EMBED_DIM), jnp.float32),
    mesh=plsc.VectorSubcoreMesh(core_axis_name="c", subcore_axis_name="s"),
    scratch_types=[
        pltpu.VMEM((CHUNK,), jnp.int32),
        pltpu.VMEM((CHUNK, EMBED_DIM), jnp.float32),
        pltpu.SemaphoreType.DMA,
    ],
)(_sc_gather_body)


@jax.jit
def kernel(x, boundaries, emb_tables):
    flat_idx = _flat_indices(x, boundaries).reshape(TOTAL)
    table = emb_tables.reshape(NUM_FEATURES * NUM_BUCKETS, EMBED_DIM)
    out = _sc_gather(table, flat_idx)
    return out.reshape(BATCH, NUM_FEATURES, EMBED_DIM)


# speedup vs baseline: 3.7290x; 3.7290x over previous
import functools

import jax
import jax.numpy as jnp
from jax import lax
from jax.experimental import pallas as pl
from jax.experimental.pallas import tpu as pltpu
from jax.experimental.pallas import tpu_sc as plsc

BATCH = 16384
NUM_FEATURES = 100
NUM_BUCKETS = 20
EMBED_DIM = 32
TOTAL = BATCH * NUM_FEATURES

NUM_WORKERS = 32
PER_W = TOTAL // NUM_WORKERS
CHUNK = 1024
NCHUNKS = PER_W // CHUNK


def _bucketize_body(x_ref, b_ref, out_ref):
    x = x_ref[...]
    x_min = jnp.min(x, axis=0, keepdims=True)
    x_max = jnp.max(x, axis=0, keepdims=True)
    xn = (x - x_min) / (x_max - x_min + 1e-08)
    bs = jax.nn.sigmoid(b_ref[...])
    cnt = (xn > 0.0).astype(jnp.int32)
    for k in range(NUM_BUCKETS - 1):
        cnt = cnt + (xn > bs[0:1, k : k + 1]).astype(jnp.int32)
    idx = jnp.clip(cnt - 1, 0, NUM_BUCKETS - 1)
    feat = lax.broadcasted_iota(jnp.int32, (BATCH, NUM_FEATURES), 1)
    out_ref[...] = idx + feat * NUM_BUCKETS


def _flat_indices(x, boundaries):
    return pl.pallas_call(
        _bucketize_body,
        out_shape=jax.ShapeDtypeStruct((BATCH, NUM_FEATURES), jnp.int32),
    )(x, boundaries.reshape(1, NUM_BUCKETS - 1))


def _sc_gather_body(table_hbm, idx_hbm, out_hbm, idx_v, rows_v, sem):
    wid = lax.axis_index("s") * 2 + lax.axis_index("c")
    base = wid * PER_W

    def body(c, carry):
        off = base + c * CHUNK
        pltpu.sync_copy(idx_hbm.at[pl.ds(off, CHUNK)], idx_v)
        pltpu.async_copy(table_hbm.at[idx_v], rows_v, sem).wait()
        pltpu.sync_copy(rows_v, out_hbm.at[pl.ds(off, CHUNK)])
        return carry

    lax.fori_loop(0, NCHUNKS, body, 0)


_sc_gather = functools.partial(
    pl.kernel,
    out_type=jax.ShapeDtypeStruct((TOTAL, EMBED_DIM), jnp.float32),
    mesh=plsc.VectorSubcoreMesh(core_axis_name="c", subcore_axis_name="s"),
    compiler_params=pltpu.CompilerParams(use_tc_tiling_on_sc=False),
    scratch_types=[
        pltpu.VMEM((CHUNK,), jnp.int32),
        pltpu.VMEM((CHUNK, EMBED_DIM), jnp.float32),
        pltpu.SemaphoreType.DMA,
    ],
)(_sc_gather_body)


@jax.jit
def kernel(x, boundaries, emb_tables):
    flat_idx = _flat_indices(x, boundaries).reshape(TOTAL)
    table = emb_tables.reshape(NUM_FEATURES * NUM_BUCKETS, EMBED_DIM)
    out = _sc_gather(table, flat_idx)
    return out.reshape(BATCH, NUM_FEATURES, EMBED_DIM)
